# baseline (speedup 1.0000x reference)
import jax
import jax.numpy as jnp
from jax.experimental import pallas as pl

N_NODES = 50000
N_EDGES = 800000
N_FEAT = 5
N_HID = 64
N_REL = 4
MAX_DEG = 10
N_GRAPHS = 32


def _relu_body(x_ref, o_ref):
    o_ref[...] = jnp.maximum(x_ref[...], 0.0)


def _relu_pallas(h):
    flat = h.reshape(25000, 128)
    out = pl.pallas_call(
        _relu_body,
        out_shape=jax.ShapeDtypeStruct((25000, 128), jnp.float32),
        grid=(25,),
        in_specs=[pl.BlockSpec((1000, 128), lambda i: (i, 0))],
        out_specs=pl.BlockSpec((1000, 128), lambda i: (i, 0)),
    )(flat)
    return out.reshape(N_NODES, N_HID)


def _rgcn(h, src, dst, edge_type, coef, weight, root, bias):
    y = jnp.einsum('nh,rhd->rnd', h, weight).reshape(N_REL * N_NODES, N_HID)
    msg = y[edge_type * N_NODES + src] * coef[:, None]
    agg = jax.ops.segment_sum(msg, dst, num_segments=N_NODES)
    return h @ root + bias + agg


def _mfconv(h, src, dst, inv_deg, degc, W_l, b_l, W_r):
    s = jax.ops.segment_sum(h[src], dst, num_segments=N_NODES)
    hm = s * inv_deg[:, None]
    out = jnp.zeros((N_NODES, N_HID), jnp.float32)
    for d in range(MAX_DEG + 1):
        mask = (degc == d).astype(jnp.float32)[:, None]
        out = out + mask * (hm @ W_l[d] + b_l[d] + h @ W_r[d])
    return out


def kernel(x, edge_index, edge_attr, batch_idx, W_emb, b_emb, rgcn0_w, rgcn0_root, rgcn0_b, mf0_wl, mf0_bl, mf0_wr, rgcn1_w, rgcn1_root, rgcn1_b, mf1_wl, mf1_bl, mf1_wr):
    src = edge_index[0]
    dst = edge_index[1]
    edge_type = jnp.argmax(edge_attr, axis=1).astype(jnp.int32)

    cnt = jax.ops.segment_sum(
        jnp.ones((N_EDGES,), jnp.float32), dst * N_REL + edge_type,
        num_segments=N_NODES * N_REL).reshape(N_NODES, N_REL)
    deg = cnt.sum(axis=1)
    degc = jnp.clip(deg.astype(jnp.int32), 0, MAX_DEG)
    inv_cnt = 1.0 / jnp.clip(cnt, 1.0)
    inv_deg = 1.0 / jnp.clip(deg, 1.0)
    coef = inv_cnt[dst, edge_type]

    h = x[:, :5] @ W_emb + b_emb
    h = _rgcn(h, src, dst, edge_type, coef, rgcn0_w, rgcn0_root, rgcn0_b)
    h = _relu_pallas(h)
    h = _mfconv(h, src, dst, inv_deg, degc, mf0_wl, mf0_bl, mf0_wr)
    h = _relu_pallas(h)
    h = _rgcn(h, src, dst, edge_type, coef, rgcn1_w, rgcn1_root, rgcn1_b)
    h = _relu_pallas(h)
    h = _mfconv(h, src, dst, inv_deg, degc, mf1_wl, mf1_bl, mf1_wr)

    ps = jax.ops.segment_sum(h, batch_idx, num_segments=N_GRAPHS)
    pc = jax.ops.segment_sum(jnp.ones((N_NODES,), jnp.float32), batch_idx, num_segments=N_GRAPHS)
    return ps / jnp.clip(pc, 1.0)[:, None]


# speedup vs baseline: 1.2703x; 1.2703x over previous
import functools
import jax
import jax.numpy as jnp
from jax import lax
from jax.experimental import pallas as pl
from jax.experimental.pallas import tpu as pltpu
from jax.experimental.pallas import tpu_sc as plsc

N_NODES = 50000
N_EDGES = 800000
N_FEAT = 5
N_HID = 64
N_REL = 4
MAX_DEG = 10
N_GRAPHS = 32

E_PAD = 819200
CHUNK = 128
STAGE = 8
ROUNDS = 50
CPT = STAGE * ROUNDS
NPT = 3128
ACC_ROWS = 16 * NPT


def _agg_body(tab_rows, table, gidx, sidx, coef, out,
              gixv, sixv, cofv, rows, dbuf, acc, gsem):
    cid = lax.axis_index("c")
    sid = lax.axis_index("s")
    zero16 = jnp.zeros((16,), jnp.float32)

    for q in range(2):
        def zrow(i, _):
            dbuf[i, pl.ds(0, 16)] = zero16
            return 0
        lax.fori_loop(0, NPT, zrow, 0)
        pltpu.sync_copy(dbuf, acc.at[pl.ds(sid * NPT, NPT)])
        plsc.subcore_barrier()

        offv = jnp.full((16,), (2 * cid + q) * tab_rows, jnp.int32)

        def round_body(rd, _):
            rb = sid * CPT + rd * STAGE
            pltpu.sync_copy(gidx.at[pl.ds(rb, STAGE)], gixv)
            pltpu.sync_copy(sidx.at[pl.ds(rb, STAGE)], sixv)
            pltpu.sync_copy(coef.at[pl.ds(rb, STAGE)], cofv)

            def chunk_body(s, _):
                for k in range(8):
                    gixv[s, pl.ds(k * 16, 16)] = gixv[s, pl.ds(k * 16, 16)] + offv
                pltpu.async_copy(table.at[gixv.at[s]], rows, gsem).wait()

                def scale(g, _):
                    cvec = cofv[s, pl.ds(g * 16, 16)]
                    for j in range(16):
                        cv = jnp.full((16,), cvec[j], jnp.float32)
                        e = g * 16 + j
                        rows[e, pl.ds(0, 16)] = rows[e, pl.ds(0, 16)] * cv
                    return 0
                lax.fori_loop(0, CHUNK // 16, scale, 0)

                pltpu.sync_copy(rows, acc.at[sixv.at[s]], add=True)
                return 0
            lax.fori_loop(0, STAGE, chunk_body, 0)
            return 0
        lax.fori_loop(0, ROUNDS, round_body, 0)
        plsc.subcore_barrier()
        pltpu.sync_copy(acc.at[pl.ds(sid * NPT, NPT)], dbuf)
        pltpu.sync_copy(
            dbuf, out.at[pl.ds((2 * cid + q) * ACC_ROWS + sid * NPT, NPT)])
        plsc.subcore_barrier()


def _sc_aggregate(tab_rows, table2, gidx2, sidx2, coef2):
    mesh = plsc.VectorSubcoreMesh(core_axis_name="c", subcore_axis_name="s")
    kfn = pl.kernel(
        functools.partial(_agg_body, tab_rows),
        out_type=pltpu.HBM((4 * ACC_ROWS, 16), jnp.float32),
        mesh=mesh,
        scratch_types=[
            pltpu.VMEM((STAGE, CHUNK), jnp.int32),
            pltpu.VMEM((STAGE, CHUNK), jnp.int32),
            pltpu.VMEM((STAGE, CHUNK), jnp.float32),
            pltpu.VMEM((CHUNK, 16), jnp.float32),
            pltpu.VMEM((NPT, 16), jnp.float32),
            pltpu.VMEM_SHARED((ACC_ROWS, 16), jnp.float32),
            pltpu.SemaphoreType.DMA,
        ],
        compiler_params=pltpu.CompilerParams(use_tc_tiling_on_sc=False),
    )
    hbm = lambda a: pltpu.with_memory_space_constraint(a, pltpu.MemorySpace.HBM)
    out = kfn(hbm(table2), hbm(gidx2), hbm(sidx2), hbm(coef2))
    return jax.device_put(out, jax.memory.Space.Device)


def _split_cols(a):
    r = a.shape[0]
    return a.reshape(r, 4, 16).transpose(1, 0, 2).reshape(4 * r, 16)


def _merge_cols(a2):
    return a2.reshape(4, ACC_ROWS, 16)[:, :N_NODES].transpose(1, 0, 2).reshape(N_NODES, 64)


def _pad_edges(v, fill):
    return jnp.concatenate(
        [v, jnp.full((E_PAD - N_EDGES,), fill, v.dtype)]).reshape(E_PAD // CHUNK, CHUNK)


def _rgcn(h, gidx2, sidx2, coef2, weight, root, bias):
    y = jnp.einsum('nh,rhd->rnd', h, weight).reshape(N_REL * N_NODES, N_HID)
    agg = _merge_cols(_sc_aggregate(N_REL * N_NODES, _split_cols(y), gidx2, sidx2, coef2))
    return h @ root + bias + agg


def _mfconv(h, gidx2, sidx2, coef2, degc, W_l, b_l, W_r):
    hm = _merge_cols(_sc_aggregate(N_NODES, _split_cols(h), gidx2, sidx2, coef2))
    out = jnp.zeros((N_NODES, N_HID), jnp.float32)
    for d in range(MAX_DEG + 1):
        mask = (degc == d).astype(jnp.float32)[:, None]
        out = out + mask * (hm @ W_l[d] + b_l[d] + h @ W_r[d])
    return out


def kernel(x, edge_index, edge_attr, batch_idx, W_emb, b_emb, rgcn0_w, rgcn0_root, rgcn0_b, mf0_wl, mf0_bl, mf0_wr, rgcn1_w, rgcn1_root, rgcn1_b, mf1_wl, mf1_bl, mf1_wr):
    src = edge_index[0]
    dst = edge_index[1]
    edge_type = jnp.argmax(edge_attr, axis=1).astype(jnp.int32)

    cnt = jax.ops.segment_sum(
        jnp.ones((N_EDGES,), jnp.float32), dst * N_REL + edge_type,
        num_segments=N_NODES * N_REL).reshape(N_NODES, N_REL)
    deg = cnt.sum(axis=1)
    degc = jnp.clip(deg.astype(jnp.int32), 0, MAX_DEG)
    inv_cnt = 1.0 / jnp.clip(cnt, 1.0)
    inv_deg = 1.0 / jnp.clip(deg, 1.0)

    gidx_r = _pad_edges(edge_type * N_NODES + src, 0)
    gidx_m = _pad_edges(src, 0)
    sidx = _pad_edges(dst, N_NODES)
    coef_r = _pad_edges(inv_cnt[dst, edge_type], 0.0)
    coef_m = _pad_edges(inv_deg[dst], 0.0)

    h = x[:, :5] @ W_emb + b_emb
    h = _rgcn(h, gidx_r, sidx, coef_r, rgcn0_w, rgcn0_root, rgcn0_b)
    h = jax.nn.relu(h)
    h = _mfconv(h, gidx_m, sidx, coef_m, degc, mf0_wl, mf0_bl, mf0_wr)
    h = jax.nn.relu(h)
    h = _rgcn(h, gidx_r, sidx, coef_r, rgcn1_w, rgcn1_root, rgcn1_b)
    h = jax.nn.relu(h)
    h = _mfconv(h, gidx_m, sidx, coef_m, degc, mf1_wl, mf1_bl, mf1_wr)

    ps = jax.ops.segment_sum(h, batch_idx, num_segments=N_GRAPHS)
    pc = jax.ops.segment_sum(jnp.ones((N_NODES,), jnp.float32), batch_idx, num_segments=N_GRAPHS)
    return ps / jnp.clip(pc, 1.0)[:, None]
